# baseline (speedup 1.0000x reference)
import functools

import jax
import jax.numpy as jnp
from jax import lax
from jax.experimental import pallas as pl
from jax.experimental.pallas import tpu as pltpu
from jax.experimental.pallas import tpu_sc as plsc

_NW = 32
_NB = 16
_C = 128 * _NB
_BB = _NB + 1


def _make_sc_pool(E, SPW, BCNT):
    NC = 2
    H = SPW // 2
    HR = H // 4
    OROWS = SPW // 4
    NBLK = E // 128
    CR = 8 * _BB
    mesh = plsc.VectorSubcoreMesh(core_axis_name="c", subcore_axis_name="s")

    @functools.partial(
        pl.kernel,
        mesh=mesh,
        compiler_params=pltpu.CompilerParams(needs_layout_passes=False),
        out_type=jax.ShapeDtypeStruct((_NW * OROWS, 128), jnp.float32),
        scratch_types=[
            pltpu.VMEM((2 * CR, 128), jnp.float32),
            pltpu.VMEM((2 * CR, 128), jnp.float32),
            pltpu.VMEM((BCNT,), jnp.int32),
            pltpu.VMEM((HR, 128), jnp.float32),
            pltpu.SemaphoreType.DMA,
            pltpu.SemaphoreType.DMA,
        ],
    )
    def pool(v_hbm, bounds_hbm, out_hbm, buf0, buf1, bvm, obuf,
             sem0, sem1):
        w = lax.axis_index("s") * NC + lax.axis_index("c")
        seg0 = w * SPW
        start8 = (seg0 // 8) * 8
        off = seg0 - start8
        pltpu.sync_copy(
            bounds_hbm.at[pl.ds(pl.multiple_of(start8, 8), BCNT)], bvm)

        def bload(i):
            return bvm[pl.ds(i, 16)][0]

        r_lo = bload(off)
        r_hi = bload(off + SPW)
        nrows = r_hi - r_lo
        nchunks = jnp.maximum((nrows + _C - 1) // _C, 1)
        nch2 = (nchunks + 1) // 2

        pos_inf = jnp.full((16,), jnp.inf, jnp.float32)
        neg_inf = jnp.full((16,), -jnp.inf, jnp.float32)
        nbits = max(1, (SPW + 1).bit_length())
        obase = pl.multiple_of(w * OROWS, 8)
        io16 = lax.iota(jnp.int32, 16)
        patt = jnp.where(io16 < 8, io16, io16 - 8 + 8 * _BB)

        def chunk_blocks(k):
            base = r_lo + k * _C
            blk0 = jnp.minimum(base // 128, NBLK - _BB)
            return base, pl.multiple_of(blk0 * 8, 8)

        def start_copy(k, buf, sem):
            _, v0 = chunk_blocks(k)
            pltpu.async_copy(
                v_hbm.at[pl.ds(v0, CR)], buf.at[pl.ds(0, CR)], sem)
            pltpu.async_copy(
                v_hbm.at[pl.ds(v0 + E // 16, CR)],
                buf.at[pl.ds(CR, CR)], sem)

        def wait_copy(k, buf, sem):
            _, v0 = chunk_blocks(k)
            pltpu.make_async_copy(
                v_hbm.at[pl.ds(v0, CR)], buf.at[pl.ds(0, CR)], sem).wait()
            pltpu.make_async_copy(
                v_hbm.at[pl.ds(v0, CR)], buf.at[pl.ds(CR, CR)], sem).wait()

        def process(k, buf, st):
            s_cur, mn, mx = st
            base, v0 = chunk_blocks(k)
            shift = base - (v0 // 8) * 128
            n = jnp.clip(r_hi - base, 0, _C)
            limit = base + n

            def bs_body(_, lohi):
                lo, hi = lohi
                mid = (lo + hi) // 2
                c = bload(off + mid) <= limit
                return jnp.where(c, mid, lo), jnp.where(c, hi, mid)

            t_max, _ = lax.fori_loop(
                0, nbits, bs_body, (jnp.int32(0), jnp.int32(SPW + 1)))

            def gather_row(x):
                b8 = (x // 128) * 8
                l = x % 128
                return plsc.load_gather(
                    buf, [patt + b8, jnp.full((16,), l, jnp.int32)])

            def row_body(j, acc):
                a, b = acc
                v = gather_row(shift + j)
                return jnp.minimum(a, v), jnp.maximum(b, v)

            def reduce_rows(i, hi, mn, mx):
                n4 = jnp.maximum(hi - i, 0) // 4

                def quad(t, acc):
                    a, b = acc
                    x = shift + i + t * 4
                    v0_ = gather_row(x)
                    v1_ = gather_row(x + 1)
                    v2_ = gather_row(x + 2)
                    v3_ = gather_row(x + 3)
                    a = jnp.minimum(
                        a, jnp.minimum(jnp.minimum(v0_, v1_),
                                       jnp.minimum(v2_, v3_)))
                    b = jnp.maximum(
                        b, jnp.maximum(jnp.maximum(v0_, v1_),
                                       jnp.maximum(v2_, v3_)))
                    return a, b

                mn, mx = lax.fori_loop(0, n4, quad, (mn, mx))
                return lax.fori_loop(i + n4 * 4, hi, row_body, (mn, mx))

            def seg_store(s_rel, st2):
                i, mn, mx = st2[0], st2[1], st2[2]
                hi_local = st2[3] - base
                mn, mx = reduce_rows(i, hi_local, mn, mx)
                obuf[s_rel // 4, pl.ds((s_rel % 4) * 32, 16)] = mn
                obuf[s_rel // 4, pl.ds((s_rel % 4) * 32 + 16, 16)] = mx
                return hi_local

            def seg_lo(s, st2):
                i, mn, mx = st2
                hi_local = seg_store(s, (i, mn, mx, bload(off + s + 1)))
                return hi_local, pos_inf, neg_inf

            def seg_hi(s, st2):
                i, mn, mx = st2
                hi_local = seg_store(s - H, (i, mn, mx, bload(off + s + 1)))
                return hi_local, pos_inf, neg_inf

            i, mn, mx = lax.fori_loop(
                s_cur, jnp.minimum(t_max, H), seg_lo,
                (jnp.int32(0), mn, mx))

            @pl.when((s_cur < H) & (t_max >= H))
            def _():
                pltpu.sync_copy(obuf, out_hbm.at[pl.ds(obase, HR)])

            i, mn, mx = lax.fori_loop(
                jnp.maximum(s_cur, H), t_max, seg_hi, (i, mn, mx))
            mn, mx = reduce_rows(i, n, mn, mx)
            return t_max, mn, mx

        start_copy(0, buf0, sem0)

        def loop_body(k2, st):
            k = 2 * k2
            start_copy(k + 1, buf1, sem1)
            wait_copy(k, buf0, sem0)
            st = process(k, buf0, st)
            start_copy(k + 2, buf0, sem0)
            wait_copy(k + 1, buf1, sem1)
            st = process(k + 1, buf1, st)
            return st

        init = (jnp.int32(0), pos_inf, neg_inf)
        lax.fori_loop(0, nch2, loop_body, init)
        wait_copy(0, buf0, sem0)
        pltpu.sync_copy(obuf, out_hbm.at[pl.ds(obase + HR, HR)])

    return pool


def kernel(edge_features, cell_0_bounds):
    E, D = edge_features.shape
    S = cell_0_bounds.shape[0] - 1
    assert D == 16 and E % 128 == 0
    SPW = ((-(-S // _NW) + 63) // 64) * 64
    S_pad = _NW * SPW
    BCNT = ((SPW + 8) // 8 + 1) * 8 + 16
    pad_len = (S_pad + 48) - (S + 1)
    bounds = jnp.concatenate(
        [cell_0_bounds.astype(jnp.int32),
         jnp.full((pad_len,), E, jnp.int32)])
    v = edge_features.reshape(E // 128, 128, 2, 8).transpose(2, 0, 3, 1)
    v = v.reshape(E // 8, 128)
    out = _make_sc_pool(E, SPW, BCNT)(v, bounds)
    return out.reshape(S_pad, 2 * D)[:S]


# speedup vs baseline: 1.2683x; 1.2683x over previous
import functools

import jax
import jax.numpy as jnp
from jax import lax
from jax.experimental import pallas as pl
from jax.experimental.pallas import tpu as pltpu
from jax.experimental.pallas import tpu_sc as plsc

_NW = 32
_NB = 16
_C = 128 * _NB
_BB = _NB + 1


def _make_sc_pool(E, SPW, BCNT):
    NC = 2
    H = SPW // 2
    HR = H // 4
    OROWS = SPW // 4
    NBLK = E // 128
    CR = 8 * _BB
    mesh = plsc.VectorSubcoreMesh(core_axis_name="c", subcore_axis_name="s")

    @functools.partial(
        pl.kernel,
        mesh=mesh,
        compiler_params=pltpu.CompilerParams(needs_layout_passes=False),
        out_type=jax.ShapeDtypeStruct((_NW * OROWS, 128), jnp.float32),
        scratch_types=[
            pltpu.VMEM((2 * CR + 8, 128), jnp.float32),
            pltpu.VMEM((2 * CR + 8, 128), jnp.float32),
            pltpu.VMEM((BCNT,), jnp.int32),
            pltpu.VMEM((HR, 128), jnp.float32),
            pltpu.VMEM((32,), jnp.float32),
            pltpu.SemaphoreType.DMA,
            pltpu.SemaphoreType.DMA,
        ],
    )
    def pool(v_hbm, bounds_hbm, out_hbm, buf0, buf1, bvm, obuf, fbuf,
             sem0, sem1):
        w = lax.axis_index("s") * NC + lax.axis_index("c")
        seg0 = w * SPW
        start8 = (seg0 // 8) * 8
        off = seg0 - start8
        pltpu.sync_copy(
            bounds_hbm.at[pl.ds(pl.multiple_of(start8, 8), BCNT)], bvm)

        def bload(i):
            return bvm[pl.ds(i, 16)][0]

        r_lo = bload(off)
        r_hi = bload(off + SPW)
        nrows = r_hi - r_lo
        nchunks = jnp.maximum((nrows + _C - 1) // _C, 1)
        nch2 = (nchunks + 1) // 2

        pos_inf = jnp.full((16,), jnp.inf, jnp.float32)
        neg_inf = jnp.full((16,), -jnp.inf, jnp.float32)
        nbits = max(1, (SPW + 1).bit_length())
        obase = pl.multiple_of(w * OROWS, 8)
        io16 = lax.iota(jnp.int32, 16)
        lo_half = io16 < 8
        pattA = io16 & 7

        def chunk_blocks(k):
            base = r_lo + k * _C
            blk0 = jnp.minimum(base // 128, NBLK - _BB)
            return base, pl.multiple_of(blk0 * 8, 8)

        def start_copy(k, buf, sem):
            _, v0 = chunk_blocks(k)
            pltpu.async_copy(
                v_hbm.at[pl.ds(v0, CR)], buf.at[pl.ds(0, CR)], sem)
            pltpu.async_copy(
                v_hbm.at[pl.ds(v0 + E // 16, CR)],
                buf.at[pl.ds(CR, CR)], sem)

        def wait_copy(k, buf, sem):
            _, v0 = chunk_blocks(k)
            pltpu.make_async_copy(
                v_hbm.at[pl.ds(v0, CR)], buf.at[pl.ds(0, CR)], sem).wait()
            pltpu.make_async_copy(
                v_hbm.at[pl.ds(v0, CR)], buf.at[pl.ds(CR, CR)], sem).wait()

        def fold_pair(accA, accB, op):
            fbuf[pl.ds(0, 16)] = accA
            fA = op(accA, fbuf[pl.ds(8, 16)])
            fbuf[pl.ds(0, 16)] = accB
            fB = op(accB, fbuf[pl.ds(8, 16)])
            fbuf[pl.ds(8, 16)] = fB
            return jnp.where(lo_half, fA, fbuf[pl.ds(0, 16)])

        def process(k, buf, st):
            s_cur, mnA, mnB, mxA, mxB = st
            base, v0 = chunk_blocks(k)
            shift = base - (v0 // 8) * 128
            n = jnp.clip(r_hi - base, 0, _C)
            limit = base + n

            def bs_body(_, lohi):
                lo, hi = lohi
                mid = (lo + hi) // 2
                c = bload(off + mid) <= limit
                return jnp.where(c, mid, lo), jnp.where(c, hi, mid)

            t_max, _ = lax.fori_loop(
                0, nbits, bs_body, (jnp.int32(0), jnp.int32(SPW + 1)))

            def reduce_rows(i, hi, accs):
                cnt = jnp.maximum(hi - i, 0)
                T = 8 * ((cnt + 15) // 16)

                def it(t, ac):
                    mnA, mnB, mxA, mxB = ac
                    x0 = i + (t // 8) * 16 + (t & 7)
                    x1 = x0 + 8
                    xs0 = shift + x0
                    xs1 = xs0 + 8
                    bvec = jnp.where(lo_half, (xs0 // 128) * 8,
                                     (xs1 // 128) * 8)
                    cols = jnp.where(lo_half, xs0 % 128, xs1 % 128)
                    mask = jnp.where(lo_half, x0 < hi, x1 < hi)
                    rowsA = pattA + bvec
                    gA = plsc.load_gather(buf, [rowsA, cols], mask=mask)
                    gB = plsc.load_gather(buf, [rowsA + CR, cols],
                                          mask=mask)
                    mnA = jnp.minimum(mnA, jnp.where(mask, gA, jnp.inf))
                    mxA = jnp.maximum(mxA, jnp.where(mask, gA, -jnp.inf))
                    mnB = jnp.minimum(mnB, jnp.where(mask, gB, jnp.inf))
                    mxB = jnp.maximum(mxB, jnp.where(mask, gB, -jnp.inf))
                    return mnA, mnB, mxA, mxB

                return lax.fori_loop(0, T, it, accs)

            def seg_store(s_rel, st2):
                i = st2[0]
                hi_local = st2[5] - base
                accs = reduce_rows(i, hi_local, st2[1:5])
                mn = fold_pair(accs[0], accs[1], jnp.minimum)
                mx = fold_pair(accs[2], accs[3], jnp.maximum)
                obuf[s_rel // 4, pl.ds((s_rel % 4) * 32, 16)] = mn
                obuf[s_rel // 4, pl.ds((s_rel % 4) * 32 + 16, 16)] = mx
                return hi_local

            def seg_lo(s, st2):
                hi_local = seg_store(s, st2 + (bload(off + s + 1),))
                return (hi_local, pos_inf, pos_inf, neg_inf, neg_inf)

            def seg_hi(s, st2):
                hi_local = seg_store(s - H, st2 + (bload(off + s + 1),))
                return (hi_local, pos_inf, pos_inf, neg_inf, neg_inf)

            st1 = lax.fori_loop(
                s_cur, jnp.minimum(t_max, H), seg_lo,
                (jnp.int32(0),) + st[1:])

            @pl.when((s_cur < H) & (t_max >= H))
            def _():
                pltpu.sync_copy(obuf, out_hbm.at[pl.ds(obase, HR)])

            st2_ = lax.fori_loop(
                jnp.maximum(s_cur, H), t_max, seg_hi, st1)
            accs = reduce_rows(st2_[0], n, st2_[1:])
            return (t_max,) + accs

        start_copy(0, buf0, sem0)

        def loop_body(k2, st):
            k = 2 * k2
            start_copy(k + 1, buf1, sem1)
            wait_copy(k, buf0, sem0)
            st = process(k, buf0, st)
            start_copy(k + 2, buf0, sem0)
            wait_copy(k + 1, buf1, sem1)
            st = process(k + 1, buf1, st)
            return st

        init = (jnp.int32(0), pos_inf, pos_inf, neg_inf, neg_inf)
        lax.fori_loop(0, nch2, loop_body, init)
        wait_copy(0, buf0, sem0)
        pltpu.sync_copy(obuf, out_hbm.at[pl.ds(obase + HR, HR)])

    return pool


def kernel(edge_features, cell_0_bounds):
    E, D = edge_features.shape
    S = cell_0_bounds.shape[0] - 1
    assert D == 16 and E % 128 == 0
    SPW = ((-(-S // _NW) + 63) // 64) * 64
    S_pad = _NW * SPW
    BCNT = ((SPW + 8) // 8 + 1) * 8 + 16
    pad_len = (S_pad + 48) - (S + 1)
    bounds = jnp.concatenate(
        [cell_0_bounds.astype(jnp.int32),
         jnp.full((pad_len,), E, jnp.int32)])
    v = edge_features.reshape(E // 128, 128, 2, 8).transpose(2, 0, 3, 1)
    v = v.reshape(E // 8, 128)
    out = _make_sc_pool(E, SPW, BCNT)(v, bounds)
    return out.reshape(S_pad, 2 * D)[:S]
